# Initial kernel scaffold; baseline (speedup 1.0000x reference)
import jax
import jax.numpy as jnp
from jax.experimental import pallas as pl


def kernel(boxes, scores):
    raise NotImplementedError("write your pallas kernel here")


# speedup vs baseline: 168.4016x; 168.4016x over previous
import jax
import jax.numpy as jnp
from jax import lax
from jax.experimental import pallas as pl
from jax.experimental.pallas import tpu as pltpu

N = 20000
NP = 20480
R = 160
C = 128
K1 = 1000
K2 = 256
S = 1024
SCORE_T = 0.05
NMS_T = 0.9
HI_BITS = 0x3F800000

_HP = lax.Precision.HIGHEST


def _excl_prefix(m):
    lane = lax.broadcasted_iota(jnp.int32, (C, C), 0)
    lane_t = lax.broadcasted_iota(jnp.int32, (C, C), 1)
    su = (lane < lane_t).astype(jnp.float32)
    within = jnp.dot(m, su, precision=_HP)
    rs = jnp.sum(m, axis=1, keepdims=True)
    ra = lax.broadcasted_iota(jnp.int32, (R, R), 0)
    rb = lax.broadcasted_iota(jnp.int32, (R, R), 1)
    sl = (rb < ra).astype(jnp.float32)
    offs = jnp.dot(sl, rs, precision=_HP)
    return within + offs


def _body(scores_ref, boxes_ref, out_ref):
    irow = lax.broadcasted_iota(jnp.int32, (S, S), 0)
    jcol = lax.broadcasted_iota(jnp.int32, (S, S), 1)
    eye = (irow == jcol).astype(jnp.float32)

    def to_row8(a8):
        b = jnp.concatenate([a8] * 8, axis=1)
        rsel = lax.broadcasted_iota(jnp.int32, (8, S), 0)
        lsel = lax.shift_right_logical(
            lax.broadcasted_iota(jnp.int32, (8, S), 1), 7)
        return jnp.sum(jnp.where(rsel == lsel, b, 0.0), axis=0, keepdims=True)

    def to_col(row):
        b = jnp.broadcast_to(row, (S, S)) * eye
        return jnp.sum(b, axis=1, keepdims=True)

    def to_row(col):
        b = jnp.broadcast_to(col, (S, S)) * eye
        return jnp.sum(b, axis=0, keepdims=True)

    def halves_col(col_f32):
        b = lax.bitcast_convert_type(col_f32, jnp.int32)
        hi = lax.shift_right_logical(b, 16)
        lo = jnp.bitwise_and(b, 0xFFFF)
        return hi.astype(jnp.float32), lo.astype(jnp.float32)

    def unhalve(hi_f, lo_f):
        hi = hi_f.astype(jnp.int32)
        lo = lo_f.astype(jnp.int32)
        return lax.bitcast_convert_type(
            jnp.bitwise_or(lax.shift_left(hi, 16), lo), jnp.float32)

    s_raw = scores_ref[...]
    s = jnp.where(s_raw > SCORE_T, s_raw, 0.0)
    t = lax.bitcast_convert_type(s, jnp.int32)

    def bs_body(_, lo_hi):
        lo, hi = lo_hi
        mid = (lo + hi) // 2
        cnt = jnp.sum(jnp.where(t >= mid, 1.0, 0.0))
        big = cnt >= K1
        return (jnp.where(big, mid, lo), jnp.where(big, hi, mid))

    lo, _ = lax.fori_loop(
        0, 31, bs_body, (jnp.int32(0), jnp.int32(HI_BITS)))
    cutv = lo

    gt = t > cutv
    eq = t == cutv
    gt_cnt = jnp.sum(gt.astype(jnp.float32))
    need = K1 - gt_cnt
    eq_pref = _excl_prefix(eq.astype(jnp.float32))
    sel = gt | (eq & (eq_pref < need))
    m = sel.astype(jnp.float32)
    pos = _excl_prefix(m)

    ii = (lax.broadcasted_iota(jnp.int32, (R, C), 0) * C
          + lax.broadcasted_iota(jnp.int32, (R, C), 1))
    d = jnp.where(sel, ii - pos.astype(jnp.int32), 0)
    act = sel.astype(jnp.int32)
    pay = [s, boxes_ref[0], boxes_ref[1], boxes_ref[2], boxes_ref[3]]

    def shift_up(a, k):
        sh = 1 << k
        if sh % C == 0:
            rsh = sh // C
            return jnp.concatenate(
                [a[rsh:, :], jnp.zeros((rsh, C), a.dtype)], axis=0)
        up = jnp.concatenate([a[1:, :], jnp.zeros((1, C), a.dtype)], axis=0)
        return jnp.concatenate([a[:, sh:], up[:, :sh]], axis=1)

    for k in range(15):
        d_s = shift_up(d, k)
        act_s = shift_up(act, k)
        bit_s = jnp.bitwise_and(lax.shift_right_logical(d_s, k), 1)
        bit = jnp.bitwise_and(lax.shift_right_logical(d, k), 1)
        inc = (act_s == 1) & (bit_s == 1)
        stay = (act == 1) & (bit == 0)
        act = (inc | stay).astype(jnp.int32)
        d = jnp.where(inc, d_s - (1 << k), jnp.where(stay, d, 0))
        pay = [jnp.where(inc, shift_up(a, k), a) for a in pay]

    a8 = act[:8, :] == 1
    s8 = jnp.where(a8, pay[0][:8, :], -1.0)
    vals8 = [s8] + [jnp.where(a8, p[:8, :], 0.0) for p in pay[1:]]

    s_row = to_row8(s8)
    s_col = to_col(s_row)
    cmp_t = ((s_col > s_row) | ((s_col == s_row) & (irow < jcol)))
    rank_row = jnp.sum(cmp_t.astype(jnp.float32), axis=0, keepdims=True)
    perm = (irow == rank_row.astype(jnp.int32)).astype(jnp.float32)

    cols = []
    for arr in vals8:
        hi8, lo8 = halves_col(arr)
        cols.append(to_col(to_row8(hi8)))
        cols.append(to_col(to_row8(lo8)))
    hmat = jnp.concatenate(cols, axis=1)
    sh_mat = jnp.dot(perm, hmat, precision=_HP)

    ss_col = unhalve(sh_mat[:, 0:1], sh_mat[:, 1:2])
    x1c = unhalve(sh_mat[:, 2:3], sh_mat[:, 3:4])
    y1c = unhalve(sh_mat[:, 4:5], sh_mat[:, 5:6])
    x2c = unhalve(sh_mat[:, 6:7], sh_mat[:, 7:8])
    y2c = unhalve(sh_mat[:, 8:9], sh_mat[:, 9:10])
    x1r, y1r = to_row(x1c), to_row(y1c)
    x2r, y2r = to_row(x2c), to_row(y2c)
    area_r = (jnp.clip(x2r - x1r, 0.0, None) * jnp.clip(y2r - y1r, 0.0, None))

    blocks = []
    for b in range(4):
        sl = slice(b * 256, (b + 1) * 256)
        x1b, y1b = x1c[sl], y1c[sl]
        x2b, y2b = x2c[sl], y2c[sl]
        inter = (jnp.clip(jnp.minimum(x2b, x2r) - jnp.maximum(x1b, x1r),
                          0.0, None)
                 * jnp.clip(jnp.minimum(y2b, y2r) - jnp.maximum(y1b, y1r),
                            0.0, None))
        area_b = (jnp.clip(x2b - x1b, 0.0, None)
                  * jnp.clip(y2b - y1b, 0.0, None))
        union = area_b + area_r - inter
        iou = inter / jnp.maximum(union, 1e-9)
        pg = lax.broadcasted_iota(jnp.int32, (256, S), 0) + b * 256
        qg = lax.broadcasted_iota(jnp.int32, (256, S), 1)
        blocks.append(((iou > NMS_T) & (qg < pg)).astype(jnp.float32))
    sup_mat = jnp.concatenate(blocks, axis=0)

    def nms_cond(carry):
        _, chg, it = carry
        return (it == 0) | (chg & (it < S + 2))

    def nms_body(carry):
        keep, _, it = carry
        supp = jnp.dot(sup_mat, keep, precision=_HP)
        keep_n = jnp.where(supp > 0.0, 0.0, 1.0)
        chg = jnp.any(keep_n != keep)
        return (keep_n, chg, it + 1)

    keep0 = jnp.ones((S, 1), jnp.float32)
    keep_col, _, _ = lax.while_loop(
        nms_cond, nms_body, (keep0, jnp.bool_(True), jnp.int32(0)))

    fs_col = ss_col * keep_col
    fs_row = to_row(fs_col)
    cmp2_t = ((fs_col > fs_row) | ((fs_col == fs_row) & (irow < jcol)))
    rank2_row = jnp.sum(cmp2_t.astype(jnp.float32), axis=0, keepdims=True)
    io2 = lax.broadcasted_iota(jnp.int32, (K2, S), 0)
    perm2 = (io2 == jnp.broadcast_to(rank2_row.astype(jnp.int32),
                                     (K2, S))).astype(jnp.float32)

    fhi, flo = halves_col(fs_col)
    h2 = jnp.concatenate([sh_mat[:, 2:10], fhi, flo], axis=1)
    o10 = jnp.dot(perm2, h2, precision=_HP)
    ox1 = unhalve(o10[:, 0:1], o10[:, 1:2])
    oy1 = unhalve(o10[:, 2:3], o10[:, 3:4])
    ox2 = unhalve(o10[:, 4:5], o10[:, 5:6])
    oy2 = unhalve(o10[:, 6:7], o10[:, 7:8])
    osc = unhalve(o10[:, 8:9], o10[:, 9:10])
    out5 = jnp.concatenate([ox1, oy1, ox2, oy2, osc], axis=1)
    out_ref[...] = jnp.concatenate(
        [out5, jnp.zeros((K2, C - 5), jnp.float32)], axis=1)


def _build(interpret=False):
    return pl.pallas_call(
        _body,
        out_shape=jax.ShapeDtypeStruct((K2, C), jnp.float32),
        interpret=interpret,
    )


def kernel(boxes, scores):
    s2 = jnp.pad(scores, (0, NP - N)).reshape(R, C)
    bt = jnp.pad(boxes, ((0, NP - N), (0, 0))).T.reshape(4, R, C)
    out = _build()(s2, bt)
    return out[:, :5]


# speedup vs baseline: 217.1326x; 1.2894x over previous
import jax
import jax.numpy as jnp
from jax import lax
from jax.experimental import pallas as pl
from jax.experimental.pallas import tpu as pltpu

N = 20000
NP = 20480
R = 160
C = 128
K1 = 1000
K2 = 256
S = 1024
SCORE_T = 0.05
NMS_T = 0.9
HI_BITS = 0x3F800000

_CNT = lax.Precision.DEFAULT


def _excl_prefix(m):
    lane = lax.broadcasted_iota(jnp.int32, (C, C), 0)
    lane_t = lax.broadcasted_iota(jnp.int32, (C, C), 1)
    su = (lane < lane_t).astype(jnp.float32)
    within = jnp.dot(m, su, precision=_CNT)
    rs = jnp.sum(m, axis=1, keepdims=True)
    ra = lax.broadcasted_iota(jnp.int32, (R, R), 0)
    rb = lax.broadcasted_iota(jnp.int32, (R, R), 1)
    sl = (rb < ra).astype(jnp.float32)
    offs = jnp.dot(sl, rs, precision=_CNT)
    return within + offs


def _body(scores_ref, boxes_ref, out_ref):
    irow = lax.broadcasted_iota(jnp.int32, (S, S), 0)
    jcol = lax.broadcasted_iota(jnp.int32, (S, S), 1)
    eye = (irow == jcol).astype(jnp.float32)

    def to_row8(a8):
        b = jnp.concatenate([a8] * 8, axis=1)
        rsel = lax.broadcasted_iota(jnp.int32, (8, S), 0)
        lsel = lax.shift_right_logical(
            lax.broadcasted_iota(jnp.int32, (8, S), 1), 7)
        return jnp.sum(jnp.where(rsel == lsel, b, 0.0), axis=0, keepdims=True)

    def to_cols(rows_mat):
        return lax.dot_general(eye, rows_mat, (((1,), (1,)), ((), ())),
                               precision=_CNT)

    def to_bytes(a):
        b = lax.bitcast_convert_type(a, jnp.int32)
        return [jnp.bitwise_and(
                    lax.shift_right_logical(b, 8 * i), 255).astype(jnp.float32)
                for i in range(4)]

    def from_bytes(bs):
        acc = bs[3].astype(jnp.int32)
        for i in (2, 1, 0):
            acc = jnp.bitwise_or(lax.shift_left(acc, 8), bs[i].astype(jnp.int32))
        return lax.bitcast_convert_type(acc, jnp.float32)

    s_raw = scores_ref[...]
    s = jnp.where(s_raw > SCORE_T, s_raw, 0.0)
    t = lax.bitcast_convert_type(s, jnp.int32)

    def bs_body(_, lo_hi):
        lo, hi = lo_hi
        mid = (lo + hi) // 2
        cnt = jnp.sum(jnp.where(t >= mid, 1.0, 0.0))
        big = cnt >= K1
        return (jnp.where(big, mid, lo), jnp.where(big, hi, mid))

    lo, _ = lax.fori_loop(
        0, 31, bs_body, (jnp.int32(0), jnp.int32(HI_BITS)))
    cutv = lo

    gt = t > cutv
    eq = t == cutv
    gt_cnt = jnp.sum(gt.astype(jnp.float32))
    need = K1 - gt_cnt
    eq_pref = _excl_prefix(eq.astype(jnp.float32))
    sel = gt | (eq & (eq_pref < need))
    m = sel.astype(jnp.float32)
    pos = _excl_prefix(m)

    ii = (lax.broadcasted_iota(jnp.int32, (R, C), 0) * C
          + lax.broadcasted_iota(jnp.int32, (R, C), 1))
    d = jnp.where(sel, ii - pos.astype(jnp.int32), 0)
    act = sel.astype(jnp.int32)
    pay = [s, boxes_ref[0], boxes_ref[1], boxes_ref[2], boxes_ref[3]]

    def shift_up(a, k):
        sh = 1 << k
        if sh % C == 0:
            rsh = sh // C
            return jnp.concatenate(
                [a[rsh:, :], jnp.zeros((rsh, C), a.dtype)], axis=0)
        up = jnp.concatenate([a[1:, :], jnp.zeros((1, C), a.dtype)], axis=0)
        return jnp.concatenate([a[:, sh:], up[:, :sh]], axis=1)

    for k in range(15):
        d_s = shift_up(d, k)
        act_s = shift_up(act, k)
        bit_s = jnp.bitwise_and(lax.shift_right_logical(d_s, k), 1)
        bit = jnp.bitwise_and(lax.shift_right_logical(d, k), 1)
        inc = (act_s == 1) & (bit_s == 1)
        stay = (act == 1) & (bit == 0)
        act = (inc | stay).astype(jnp.int32)
        d = jnp.where(inc, d_s - (1 << k), jnp.where(stay, d, 0))
        pay = [jnp.where(inc, shift_up(a, k), a) for a in pay]

    a8 = act[:8, :] == 1
    s8 = jnp.where(a8, pay[0][:8, :], -1.0)
    vals8 = [s8] + [jnp.where(a8, p[:8, :], 0.0) for p in pay[1:]]

    s_row = to_row8(s8)
    sbytes_rows = jnp.concatenate([to_row8(b) for b in to_bytes(s8)], axis=0)
    sb_cols = to_cols(sbytes_rows)
    s_col = from_bytes([sb_cols[:, i:i + 1] for i in range(4)])
    cmp_t = ((s_col > s_row) | ((s_col == s_row) & (irow < jcol)))
    ones_row = jnp.ones((1, S), jnp.float32)
    rank_row = jnp.dot(ones_row, cmp_t.astype(jnp.float32), precision=_CNT)
    perm = (irow == rank_row.astype(jnp.int32)).astype(jnp.float32)

    rows = []
    for arr in vals8:
        rows.extend(to_row8(b) for b in to_bytes(arr))
    hmat_t = jnp.concatenate(rows, axis=0)
    sh_mat = lax.dot_general(perm, hmat_t, (((1,), (1,)), ((), ())),
                             precision=_CNT)
    sh_mat_t = lax.dot_general(hmat_t, perm, (((1,), (1,)), ((), ())),
                               precision=_CNT)

    def col_val(c):
        return from_bytes([sh_mat[:, 4 * c + i:4 * c + i + 1]
                           for i in range(4)])

    def row_val(c):
        return from_bytes([sh_mat_t[4 * c + i:4 * c + i + 1, :]
                           for i in range(4)])

    ss_col, x1c, y1c, x2c, y2c = [col_val(c) for c in range(5)]
    ss_row, x1r, y1r, x2r, y2r = [row_val(c) for c in range(5)]
    area_r = (jnp.clip(x2r - x1r, 0.0, None) * jnp.clip(y2r - y1r, 0.0, None))

    blocks = []
    for b in range(4):
        sl = slice(b * 256, (b + 1) * 256)
        x1b, y1b = x1c[sl], y1c[sl]
        x2b, y2b = x2c[sl], y2c[sl]
        inter = (jnp.clip(jnp.minimum(x2b, x2r) - jnp.maximum(x1b, x1r),
                          0.0, None)
                 * jnp.clip(jnp.minimum(y2b, y2r) - jnp.maximum(y1b, y1r),
                            0.0, None))
        area_b = (jnp.clip(x2b - x1b, 0.0, None)
                  * jnp.clip(y2b - y1b, 0.0, None))
        union = area_b + area_r - inter
        iou = inter / jnp.maximum(union, 1e-9)
        qg = lax.broadcasted_iota(jnp.int32, (256, S), 0) + b * 256
        pg = lax.broadcasted_iota(jnp.int32, (256, S), 1)
        blocks.append(((iou > NMS_T) & (qg < pg)).astype(jnp.float32))
    sup_t = jnp.concatenate(blocks, axis=0)

    def nms_cond(carry):
        _, chg, it = carry
        return (it == 0) | (chg & (it < S + 2))

    def nms_body(carry):
        keep, _, it = carry
        supp = jnp.dot(keep, sup_t, precision=_CNT)
        keep_n = jnp.where(supp > 0.0, 0.0, 1.0)
        chg = jnp.any(keep_n != keep)
        return (keep_n, chg, it + 1)

    keep0 = jnp.ones((1, S), jnp.float32)
    keep_row, _, _ = lax.while_loop(
        nms_cond, nms_body, (keep0, jnp.bool_(True), jnp.int32(0)))

    fs_row = ss_row * keep_row
    fb_rows = jnp.concatenate(to_bytes(fs_row), axis=0)
    fb_cols = to_cols(fb_rows)
    fs_col = from_bytes([fb_cols[:, i:i + 1] for i in range(4)])
    cmp2_t = ((fs_col > fs_row) | ((fs_col == fs_row) & (irow < jcol)))
    rank2_row = jnp.dot(ones_row, cmp2_t.astype(jnp.float32), precision=_CNT)
    io2 = lax.broadcasted_iota(jnp.int32, (K2, S), 0)
    perm2 = (io2 == jnp.broadcast_to(rank2_row.astype(jnp.int32),
                                     (K2, S))).astype(jnp.float32)

    h2_t = jnp.concatenate([sh_mat_t[4:20], fb_rows], axis=0)
    o20 = lax.dot_general(perm2, h2_t, (((1,), (1,)), ((), ())),
                          precision=_CNT)
    outs = [from_bytes([o20[:, 4 * c + i:4 * c + i + 1] for i in range(4)])
            for c in range(5)]
    out5 = jnp.concatenate(outs, axis=1)
    out_ref[...] = jnp.concatenate(
        [out5, jnp.zeros((K2, C - 5), jnp.float32)], axis=1)


def _build(interpret=False):
    return pl.pallas_call(
        _body,
        out_shape=jax.ShapeDtypeStruct((K2, C), jnp.float32),
        interpret=interpret,
    )


def kernel(boxes, scores):
    s2 = jnp.pad(scores, (0, NP - N)).reshape(R, C)
    bt = jnp.pad(boxes, ((0, NP - N), (0, 0))).T.reshape(4, R, C)
    out = _build()(s2, bt)
    return out[:, :5]


# speedup vs baseline: 223.0493x; 1.0272x over previous
import jax
import jax.numpy as jnp
from jax import lax
from jax.experimental import pallas as pl
from jax.experimental.pallas import tpu as pltpu

N = 20000
NP = 20480
R = 160
C = 128
K1 = 1000
K2 = 256
S = 1024
SCORE_T = 0.05
NMS_T = 0.9
HI_BITS = 0x3F800000
LO_BITS = 0x3D4CCCCE

_CNT = lax.Precision.DEFAULT


def _excl_prefix(m):
    lane = lax.broadcasted_iota(jnp.int32, (C, C), 0)
    lane_t = lax.broadcasted_iota(jnp.int32, (C, C), 1)
    su = (lane < lane_t).astype(jnp.float32)
    within = jnp.dot(m, su, precision=_CNT)
    rs = jnp.sum(m, axis=1, keepdims=True)
    ra = lax.broadcasted_iota(jnp.int32, (R, R), 0)
    rb = lax.broadcasted_iota(jnp.int32, (R, R), 1)
    sl = (rb < ra).astype(jnp.float32)
    offs = jnp.dot(sl, rs, precision=_CNT)
    return within + offs


def _body(scores_ref, boxes_ref, out_ref):
    irow = lax.broadcasted_iota(jnp.int32, (S, S), 0)
    jcol = lax.broadcasted_iota(jnp.int32, (S, S), 1)
    eye = (irow == jcol).astype(jnp.float32)

    def to_row8(a8):
        b = jnp.concatenate([a8] * 8, axis=1)
        rsel = lax.broadcasted_iota(jnp.int32, (8, S), 0)
        lsel = lax.shift_right_logical(
            lax.broadcasted_iota(jnp.int32, (8, S), 1), 7)
        return jnp.sum(jnp.where(rsel == lsel, b, 0.0), axis=0, keepdims=True)

    def to_cols(rows_mat):
        return lax.dot_general(eye, rows_mat, (((1,), (1,)), ((), ())),
                               precision=_CNT)

    def to_bytes(a):
        b = lax.bitcast_convert_type(a, jnp.int32)
        return [jnp.bitwise_and(
                    lax.shift_right_logical(b, 8 * i), 255).astype(jnp.float32)
                for i in range(4)]

    def from_bytes(bs):
        acc = bs[3].astype(jnp.int32)
        for i in (2, 1, 0):
            acc = jnp.bitwise_or(lax.shift_left(acc, 8), bs[i].astype(jnp.int32))
        return lax.bitcast_convert_type(acc, jnp.float32)

    s_raw = scores_ref[...]
    s = jnp.where(s_raw > SCORE_T, s_raw, 0.0)
    t = lax.bitcast_convert_type(s, jnp.int32)

    def bs_body(_, lo_hi):
        lo, hi = lo_hi
        mid = (lo + hi) // 2
        cnt = jnp.sum(jnp.where(t >= mid, 1.0, 0.0))
        big = cnt >= K1
        return (jnp.where(big, mid, lo), jnp.where(big, hi, mid))

    cnt_l = jnp.sum(jnp.where(t >= LO_BITS, 1.0, 0.0))
    big_l = cnt_l >= K1
    lo0 = jnp.where(big_l, jnp.int32(LO_BITS), jnp.int32(0))
    hi0 = jnp.where(big_l, jnp.int32(HI_BITS), jnp.int32(LO_BITS))
    lo, _ = lax.fori_loop(0, 26, bs_body, (lo0, hi0))
    cutv = lo

    gt = t > cutv
    eq = t == cutv
    gt_cnt = jnp.sum(gt.astype(jnp.float32))
    need = K1 - gt_cnt
    eq_pref = _excl_prefix(eq.astype(jnp.float32))
    sel = gt | (eq & (eq_pref < need))
    m = sel.astype(jnp.float32)
    pos = _excl_prefix(m)

    ii = (lax.broadcasted_iota(jnp.int32, (R, C), 0) * C
          + lax.broadcasted_iota(jnp.int32, (R, C), 1))
    d = jnp.where(sel, ii - pos.astype(jnp.int32), 0)
    act = sel.astype(jnp.int32)
    pay = [s, boxes_ref[0], boxes_ref[1], boxes_ref[2], boxes_ref[3]]

    def shift_up(a, k):
        sh = 1 << k
        if sh % C == 0:
            rsh = sh // C
            return jnp.concatenate(
                [a[rsh:, :], jnp.zeros((rsh, C), a.dtype)], axis=0)
        up = jnp.concatenate([a[1:, :], jnp.zeros((1, C), a.dtype)], axis=0)
        return jnp.concatenate([a[:, sh:], up[:, :sh]], axis=1)

    for k in range(15):
        d_s = shift_up(d, k)
        act_s = shift_up(act, k)
        bit_s = jnp.bitwise_and(lax.shift_right_logical(d_s, k), 1)
        bit = jnp.bitwise_and(lax.shift_right_logical(d, k), 1)
        inc = (act_s == 1) & (bit_s == 1)
        stay = (act == 1) & (bit == 0)
        act = (inc | stay).astype(jnp.int32)
        d = jnp.where(inc, d_s - (1 << k), jnp.where(stay, d, 0))
        pay = [jnp.where(inc, shift_up(a, k), a) for a in pay]

    a8 = act[:8, :] == 1
    s8 = jnp.where(a8, pay[0][:8, :], -1.0)
    vals8 = [s8] + [jnp.where(a8, p[:8, :], 0.0) for p in pay[1:]]

    s_row = to_row8(s8)
    sbytes_rows = jnp.concatenate([to_row8(b) for b in to_bytes(s8)], axis=0)
    sb_cols = to_cols(sbytes_rows)
    s_col = from_bytes([sb_cols[:, i:i + 1] for i in range(4)])
    cmp_t = ((s_col > s_row) | ((s_col == s_row) & (irow < jcol)))
    ones_row = jnp.ones((1, S), jnp.float32)
    rank_row = jnp.dot(ones_row, cmp_t.astype(jnp.float32), precision=_CNT)
    perm = (irow == rank_row.astype(jnp.int32)).astype(jnp.float32)

    rows = []
    for arr in vals8:
        rows.extend(to_row8(b) for b in to_bytes(arr))
    hmat_t = jnp.concatenate(rows, axis=0)
    sh_mat = lax.dot_general(perm, hmat_t, (((1,), (1,)), ((), ())),
                             precision=_CNT)
    sh_mat_t = lax.dot_general(hmat_t, perm, (((1,), (1,)), ((), ())),
                               precision=_CNT)

    def col_val(c):
        return from_bytes([sh_mat[:, 4 * c + i:4 * c + i + 1]
                           for i in range(4)])

    def row_val(c):
        return from_bytes([sh_mat_t[4 * c + i:4 * c + i + 1, :]
                           for i in range(4)])

    ss_col, x1c, y1c, x2c, y2c = [col_val(c) for c in range(5)]
    ss_row, x1r, y1r, x2r, y2r = [row_val(c) for c in range(5)]
    area_r = (jnp.clip(x2r - x1r, 0.0, None) * jnp.clip(y2r - y1r, 0.0, None))

    blocks = []
    for b in range(4):
        sl = slice(b * 256, (b + 1) * 256)
        x1b, y1b = x1c[sl], y1c[sl]
        x2b, y2b = x2c[sl], y2c[sl]
        inter = (jnp.clip(jnp.minimum(x2b, x2r) - jnp.maximum(x1b, x1r),
                          0.0, None)
                 * jnp.clip(jnp.minimum(y2b, y2r) - jnp.maximum(y1b, y1r),
                            0.0, None))
        area_b = (jnp.clip(x2b - x1b, 0.0, None)
                  * jnp.clip(y2b - y1b, 0.0, None))
        union = area_b + area_r - inter
        iou = inter / jnp.maximum(union, 1e-9)
        qg = lax.broadcasted_iota(jnp.int32, (256, S), 0) + b * 256
        pg = lax.broadcasted_iota(jnp.int32, (256, S), 1)
        blocks.append(((iou > NMS_T) & (qg < pg)).astype(jnp.float32))
    sup_t = jnp.concatenate(blocks, axis=0)

    def nms_cond(carry):
        _, chg, it = carry
        return (it == 0) | (chg & (it < S + 2))

    def nms_body(carry):
        keep, _, it = carry
        supp = jnp.dot(keep, sup_t, precision=_CNT)
        keep_n = jnp.where(supp > 0.0, 0.0, 1.0)
        chg = jnp.any(keep_n != keep)
        return (keep_n, chg, it + 1)

    keep0 = jnp.ones((1, S), jnp.float32)
    keep_row, _, _ = lax.while_loop(
        nms_cond, nms_body, (keep0, jnp.bool_(True), jnp.int32(0)))

    fs_row = ss_row * keep_row
    fb_rows = jnp.concatenate(to_bytes(fs_row), axis=0)
    fb_cols = to_cols(fb_rows)
    fs_col = from_bytes([fb_cols[:, i:i + 1] for i in range(4)])
    cmp2_t = ((fs_col > fs_row) | ((fs_col == fs_row) & (irow < jcol)))
    rank2_row = jnp.dot(ones_row, cmp2_t.astype(jnp.float32), precision=_CNT)
    io2 = lax.broadcasted_iota(jnp.int32, (K2, S), 0)
    perm2 = (io2 == jnp.broadcast_to(rank2_row.astype(jnp.int32),
                                     (K2, S))).astype(jnp.float32)

    h2_t = jnp.concatenate([sh_mat_t[4:20], fb_rows], axis=0)
    o20 = lax.dot_general(perm2, h2_t, (((1,), (1,)), ((), ())),
                          precision=_CNT)
    outs = [from_bytes([o20[:, 4 * c + i:4 * c + i + 1] for i in range(4)])
            for c in range(5)]
    out5 = jnp.concatenate(outs, axis=1)
    out_ref[...] = jnp.concatenate(
        [out5, jnp.zeros((K2, C - 5), jnp.float32)], axis=1)


def _build(interpret=False):
    return pl.pallas_call(
        _body,
        out_shape=jax.ShapeDtypeStruct((K2, C), jnp.float32),
        interpret=interpret,
    )


def kernel(boxes, scores):
    s2 = jnp.pad(scores, (0, NP - N)).reshape(R, C)
    bt = jnp.pad(boxes, ((0, NP - N), (0, 0))).T.reshape(4, R, C)
    out = _build()(s2, bt)
    return out[:, :5]
